# Initial kernel scaffold; baseline (speedup 1.0000x reference)
import jax
import jax.numpy as jnp
from jax.experimental import pallas as pl


def kernel(x, adj_t):
    raise NotImplementedError("write your pallas kernel here")


# speedup vs baseline: 1.8198x; 1.8198x over previous
import jax
import jax.numpy as jnp
from jax.experimental import pallas as pl

N = 4096
NV = 16
BI = 512
G = N // BI
NKEYS = 9


def _counts_body(adj_ref, onehot_ref, counts_ref, colsum_ref):
    i = pl.program_id(0)
    adj = adj_ref[...]
    oh = onehot_ref[...]
    counts_ref[...] = jnp.dot(adj, oh, preferred_element_type=jnp.float32)
    part = jnp.sum(adj, axis=0, keepdims=True)

    @pl.when(i == 0)
    def _():
        colsum_ref[...] = part

    @pl.when(i != 0)
    def _():
        colsum_ref[...] = colsum_ref[...] + part


def _keys_body(counts_ref, onehot_ref, colsum_t_ref, keys_ref, iso_ref):
    c = counts_ref[...].astype(jnp.int32)
    oh = onehot_ref[...]
    rowsum = jnp.sum(c, axis=1, keepdims=True)
    colsum = colsum_t_ref[...].astype(jnp.int32)
    iso = (rowsum + colsum) == 0
    lane = jax.lax.broadcasted_iota(jnp.int32, (N, NV), 1)
    xval = jnp.sum(oh.astype(jnp.int32) * lane, axis=1, keepdims=True)
    cols = [iso.astype(jnp.int32) * NV + xval]
    for k in range(8):
        cols.append(c[:, 2 * k : 2 * k + 1] * 8192 + c[:, 2 * k + 1 : 2 * k + 2])
    cols.append(jnp.zeros((N, NV - NKEYS), jnp.int32))
    keys_ref[...] = jnp.concatenate(cols, axis=1)
    iso_ref[...] = iso.astype(jnp.int32)


def _firstocc_body(keys_ref, keysT_ref, fo_ref):
    kb = keys_ref[...]
    kT = keysT_ref[...]
    eq = kb[:, 0:1] == kT[0:1, :]
    for k in range(1, NKEYS):
        eq = eq & (kb[:, k : k + 1] == kT[k : k + 1, :])
    idxj = jax.lax.broadcasted_iota(jnp.int32, (BI, N), 1)
    cand = jnp.where(eq, idxj, jnp.int32(N))
    fo_ref[...] = jnp.min(cand, axis=1, keepdims=True)


def _colors_body(fo_ref, iso_ref, foT_ref, isoT_ref, colors_ref):
    fo = fo_ref[...]
    iso = iso_ref[...]
    foT = foT_ref[...]
    isoT = isoT_ref[...]
    lane = jax.lax.broadcasted_iota(jnp.int32, (1, N), 1)
    is_first = (foT == lane) & (isoT == 0)
    le = jax.lax.broadcasted_iota(jnp.int32, (BI, N), 1) <= fo
    ranks = jnp.sum((le & is_first).astype(jnp.int32), axis=1, keepdims=True)
    colors_ref[...] = jnp.where(iso == 1, 0, ranks)


def kernel(x, adj_t):
    x32 = x.astype(jnp.int32).reshape(N, 1)
    onehot = (x32 == jnp.arange(NV, dtype=jnp.int32)[None, :]).astype(jnp.float32)

    counts, colsum = pl.pallas_call(
        _counts_body,
        grid=(G,),
        in_specs=[
            pl.BlockSpec((BI, N), lambda i: (i, 0)),
            pl.BlockSpec((N, NV), lambda i: (0, 0)),
        ],
        out_specs=[
            pl.BlockSpec((BI, NV), lambda i: (i, 0)),
            pl.BlockSpec((1, N), lambda i: (0, 0)),
        ],
        out_shape=[
            jax.ShapeDtypeStruct((N, NV), jnp.float32),
            jax.ShapeDtypeStruct((1, N), jnp.float32),
        ],
    )(adj_t, onehot)

    colsum_t = colsum.reshape(N, 1)
    keys, iso = pl.pallas_call(
        _keys_body,
        in_specs=[
            pl.BlockSpec((N, NV), lambda: (0, 0)),
            pl.BlockSpec((N, NV), lambda: (0, 0)),
            pl.BlockSpec((N, 1), lambda: (0, 0)),
        ],
        out_specs=[
            pl.BlockSpec((N, NV), lambda: (0, 0)),
            pl.BlockSpec((N, 1), lambda: (0, 0)),
        ],
        out_shape=[
            jax.ShapeDtypeStruct((N, NV), jnp.int32),
            jax.ShapeDtypeStruct((N, 1), jnp.int32),
        ],
    )(counts, onehot, colsum_t)

    keysT = keys.T
    fo = pl.pallas_call(
        _firstocc_body,
        grid=(G,),
        in_specs=[
            pl.BlockSpec((BI, NV), lambda i: (i, 0)),
            pl.BlockSpec((NV, N), lambda i: (0, 0)),
        ],
        out_specs=pl.BlockSpec((BI, 1), lambda i: (i, 0)),
        out_shape=jax.ShapeDtypeStruct((N, 1), jnp.int32),
    )(keys, keysT)

    foT = fo.reshape(1, N)
    isoT = iso.reshape(1, N)
    colors = pl.pallas_call(
        _colors_body,
        grid=(G,),
        in_specs=[
            pl.BlockSpec((BI, 1), lambda i: (i, 0)),
            pl.BlockSpec((BI, 1), lambda i: (i, 0)),
            pl.BlockSpec((1, N), lambda i: (0, 0)),
            pl.BlockSpec((1, N), lambda i: (0, 0)),
        ],
        out_specs=pl.BlockSpec((BI, 1), lambda i: (i, 0)),
        out_shape=jax.ShapeDtypeStruct((N, 1), jnp.int32),
    )(fo, iso, foT, isoT)

    return colors.reshape(N).astype(jnp.int64)


# speedup vs baseline: 2.8803x; 1.5828x over previous
import jax
import jax.numpy as jnp
from jax.experimental import pallas as pl

N = 4096
NV = 16
BI = 512
G = N // BI
BJ = 1024
GJ = N // BJ
SD = 64


def _counts_body(adj_ref, onehot_ref, counts_ref, colsum_ref):
    i = pl.program_id(0)
    adj = adj_ref[...]
    oh = onehot_ref[...]
    counts_ref[...] = jnp.dot(adj, oh, preferred_element_type=jnp.float32)
    part = jnp.sum(adj, axis=0, keepdims=True)

    @pl.when(i == 0)
    def _():
        colsum_ref[...] = part

    @pl.when(i != 0)
    def _():
        colsum_ref[...] = colsum_ref[...] + part


def _sig_body(counts_ref, onehot_ref, colsum_t_ref, sig_ref, nrm_ref, iso_ref):
    c = counts_ref[...].astype(jnp.int32)
    oh = onehot_ref[...]
    rowsum = jnp.sum(c, axis=1, keepdims=True)
    colsum = colsum_t_ref[...].astype(jnp.int32)
    iso = ((rowsum + colsum) == 0).astype(jnp.int32)
    lane = jax.lax.broadcasted_iota(jnp.int32, (N, NV), 1)
    xval = jnp.sum(oh.astype(jnp.int32) * lane, axis=1, keepdims=True)
    hi = c >> 6
    lo = c & 63
    sig = jnp.concatenate(
        [
            iso.astype(jnp.float32),
            xval.astype(jnp.float32),
            hi.astype(jnp.float32),
            lo.astype(jnp.float32),
            jnp.zeros((N, SD - 2 - 2 * NV), jnp.float32),
        ],
        axis=1,
    )
    sig_ref[...] = sig.astype(jnp.bfloat16)
    nrm_ref[...] = jnp.sum(sig * sig, axis=1, keepdims=True)
    iso_ref[...] = iso


def _firstocc_body(sig_ref, sig2T_ref, nrm_ref, nrmT_ref, fo_ref):
    i = pl.program_id(0)
    j = pl.program_id(1)

    @pl.when(j == 0)
    def _():
        fo_ref[...] = jnp.full((BI, 1), N, jnp.int32)

    @pl.when(j * BJ <= i * BI)
    def _():
        g2 = jnp.dot(
            sig_ref[...], sig2T_ref[...], preferred_element_type=jnp.float32
        )
        nsum = nrm_ref[...] + nrmT_ref[...]
        idxj = jax.lax.broadcasted_iota(jnp.int32, (BI, BJ), 1) + j * BJ
        cand = jnp.where(g2 == nsum, idxj, jnp.int32(N))
        fo_ref[...] = jnp.minimum(
            fo_ref[...], jnp.min(cand, axis=1, keepdims=True)
        )


def _colors_body(fo_ref, iso_ref, foT_ref, isoT_ref, colors_ref):
    fo = fo_ref[...]
    iso = iso_ref[...]
    foT = foT_ref[...]
    isoT = isoT_ref[...]
    lane = jax.lax.broadcasted_iota(jnp.int32, (1, N), 1)
    is_first = (foT == lane) & (isoT == 0)
    le = jax.lax.broadcasted_iota(jnp.int32, (BI, N), 1) <= fo
    ranks = jnp.sum((le & is_first).astype(jnp.int32), axis=1, keepdims=True)
    colors_ref[...] = jnp.where(iso == 1, 0, ranks)


def kernel(x, adj_t):
    x32 = x.astype(jnp.int32).reshape(N, 1)
    onehot = (x32 == jnp.arange(NV, dtype=jnp.int32)[None, :]).astype(jnp.float32)

    counts, colsum = pl.pallas_call(
        _counts_body,
        grid=(G,),
        in_specs=[
            pl.BlockSpec((BI, N), lambda i: (i, 0)),
            pl.BlockSpec((N, NV), lambda i: (0, 0)),
        ],
        out_specs=[
            pl.BlockSpec((BI, NV), lambda i: (i, 0)),
            pl.BlockSpec((1, N), lambda i: (0, 0)),
        ],
        out_shape=[
            jax.ShapeDtypeStruct((N, NV), jnp.float32),
            jax.ShapeDtypeStruct((1, N), jnp.float32),
        ],
    )(adj_t, onehot)

    colsum_t = colsum.reshape(N, 1)
    sig, nrm, iso = pl.pallas_call(
        _sig_body,
        in_specs=[
            pl.BlockSpec((N, NV), lambda: (0, 0)),
            pl.BlockSpec((N, NV), lambda: (0, 0)),
            pl.BlockSpec((N, 1), lambda: (0, 0)),
        ],
        out_specs=[
            pl.BlockSpec((N, SD), lambda: (0, 0)),
            pl.BlockSpec((N, 1), lambda: (0, 0)),
            pl.BlockSpec((N, 1), lambda: (0, 0)),
        ],
        out_shape=[
            jax.ShapeDtypeStruct((N, SD), jnp.bfloat16),
            jax.ShapeDtypeStruct((N, 1), jnp.float32),
            jax.ShapeDtypeStruct((N, 1), jnp.int32),
        ],
    )(counts, onehot, colsum_t)

    sig2T = (sig * jnp.bfloat16(2)).T
    nrmT = nrm.reshape(1, N)
    fo = pl.pallas_call(
        _firstocc_body,
        grid=(G, GJ),
        in_specs=[
            pl.BlockSpec((BI, SD), lambda i, j: (i, 0)),
            pl.BlockSpec((SD, BJ), lambda i, j: (0, j)),
            pl.BlockSpec((BI, 1), lambda i, j: (i, 0)),
            pl.BlockSpec((1, BJ), lambda i, j: (0, j)),
        ],
        out_specs=pl.BlockSpec((BI, 1), lambda i, j: (i, 0)),
        out_shape=jax.ShapeDtypeStruct((N, 1), jnp.int32),
    )(sig, sig2T, nrm, nrmT)

    foT = fo.reshape(1, N)
    isoT = iso.reshape(1, N)
    colors = pl.pallas_call(
        _colors_body,
        grid=(G,),
        in_specs=[
            pl.BlockSpec((BI, 1), lambda i: (i, 0)),
            pl.BlockSpec((BI, 1), lambda i: (i, 0)),
            pl.BlockSpec((1, N), lambda i: (0, 0)),
            pl.BlockSpec((1, N), lambda i: (0, 0)),
        ],
        out_specs=pl.BlockSpec((BI, 1), lambda i: (i, 0)),
        out_shape=jax.ShapeDtypeStruct((N, 1), jnp.int32),
    )(fo, iso, foT, isoT)

    return colors.reshape(N).astype(jnp.int64)


# speedup vs baseline: 2.9633x; 1.0288x over previous
import jax
import jax.numpy as jnp
from jax.experimental import pallas as pl
from jax.experimental.pallas import tpu as pltpu

N = 4096
NV = 16
BI = 512
G = N // BI
BJ = 1024
GJ = N // BJ
SD = 40
TSTEPS = G + 1 + G


def _fused_body(adj_ref, oh_ref, iso_ref, fo_ref, counts_ref, colsum_ref,
                t_ref, u_ref):
    t = pl.program_id(0)
    j = pl.program_id(1)

    @pl.when((t < G) & (j == 0))
    def _():
        adj = adj_ref[...]
        counts_ref[pl.ds(t * BI, BI), :] = jnp.dot(
            adj, oh_ref[...], preferred_element_type=jnp.float32
        )
        part = jnp.sum(adj, axis=0, keepdims=True)

        @pl.when(t == 0)
        def _():
            colsum_ref[...] = part

        @pl.when(t != 0)
        def _():
            colsum_ref[...] = colsum_ref[...] + part

    @pl.when((t == G) & (j == 0))
    def _():
        eye = (
            jax.lax.broadcasted_iota(jnp.int32, (BI, BI), 0)
            == jax.lax.broadcasted_iota(jnp.int32, (BI, BI), 1)
        ).astype(jnp.float32)
        cs_rows = [
            jax.lax.dot_general(
                eye,
                colsum_ref[:, pl.ds(b * BI, BI)],
                (((1,), (1,)), ((), ())),
                preferred_element_type=jnp.float32,
            )
            for b in range(G)
        ]
        colsum_row = jnp.concatenate(cs_rows, axis=0).astype(jnp.int32)

        c = counts_ref[...].astype(jnp.int32)
        rowsum = jnp.sum(c, axis=1, keepdims=True)
        iso = ((rowsum + colsum_row) == 0).astype(jnp.int32)
        lane = jax.lax.broadcasted_iota(jnp.int32, (N, NV), 1)
        oh_i = oh_ref[...].astype(jnp.int32)
        xval = jnp.sum(oh_i * lane, axis=1, keepdims=True)
        hi = c >> 6
        lo = c & 63
        nrm = iso + xval * xval + jnp.sum(hi * hi + lo * lo, axis=1,
                                          keepdims=True)
        n0 = (nrm & 63).astype(jnp.bfloat16)
        n1 = ((nrm >> 6) & 63).astype(jnp.bfloat16) * jnp.bfloat16(64)
        n2 = (nrm >> 12).astype(jnp.bfloat16) * jnp.bfloat16(4096)
        isob = iso.astype(jnp.bfloat16)
        xb = xval.astype(jnp.bfloat16)
        hib = hi.astype(jnp.bfloat16)
        lob = lo.astype(jnp.bfloat16)
        ones3 = jnp.ones((N, 3), jnp.bfloat16)
        two = jnp.bfloat16(2)
        t_ref[...] = jnp.concatenate(
            [isob * two, xb * two, hib * two, lob * two, -n0, -n1, -n2, ones3],
            axis=1,
        )
        u_ref[...] = jnp.concatenate(
            [isob, xb, hib, lob, ones3, -n0, -n1, -n2], axis=1
        )
        iso_ref[...] = iso

    @pl.when(t > G)
    def _():
        i = t - G - 1

        @pl.when(j == 0)
        def _():
            fo_ref[...] = jnp.full((BI, 1), N, jnp.int32)

        @pl.when(j * BJ <= i * BI)
        def _():
            g2 = jax.lax.dot_general(
                t_ref[pl.ds(i * BI, BI), :],
                u_ref[pl.ds(j * BJ, BJ), :],
                (((1,), (1,)), ((), ())),
                preferred_element_type=jnp.float32,
            )
            idxj = jax.lax.broadcasted_iota(jnp.int32, (BI, BJ), 1) + j * BJ
            cand = jnp.where(g2 == 0.0, idxj, jnp.int32(N))
            fo_ref[...] = jnp.minimum(
                fo_ref[...], jnp.min(cand, axis=1, keepdims=True)
            )


def _colors_body(fo_ref, iso_ref, foT_ref, isoT_ref, colors_ref):
    fo = fo_ref[...]
    iso = iso_ref[...]
    foT = foT_ref[...]
    isoT = isoT_ref[...]
    lane = jax.lax.broadcasted_iota(jnp.int32, (1, N), 1)
    is_first = (foT == lane) & (isoT == 0)
    le = jax.lax.broadcasted_iota(jnp.int32, (BI, N), 1) <= fo
    ranks = jnp.sum((le & is_first).astype(jnp.int32), axis=1, keepdims=True)
    colors_ref[...] = jnp.where(iso == 1, 0, ranks)


def kernel(x, adj_t):
    x32 = x.astype(jnp.int32).reshape(N, 1)
    onehot = (x32 == jnp.arange(NV, dtype=jnp.int32)[None, :]).astype(jnp.float32)

    iso, fo = pl.pallas_call(
        _fused_body,
        grid=(TSTEPS, GJ),
        in_specs=[
            pl.BlockSpec((BI, N), lambda t, j: (jnp.minimum(t, G - 1), 0)),
            pl.BlockSpec((N, NV), lambda t, j: (0, 0)),
        ],
        out_specs=[
            pl.BlockSpec((N, 1), lambda t, j: (0, 0)),
            pl.BlockSpec(
                (BI, 1),
                lambda t, j: (jnp.clip(t - G - 1, 0, G - 1), 0),
            ),
        ],
        out_shape=[
            jax.ShapeDtypeStruct((N, 1), jnp.int32),
            jax.ShapeDtypeStruct((N, 1), jnp.int32),
        ],
        scratch_shapes=[
            pltpu.VMEM((N, NV), jnp.float32),
            pltpu.VMEM((1, N), jnp.float32),
            pltpu.VMEM((N, SD), jnp.bfloat16),
            pltpu.VMEM((N, SD), jnp.bfloat16),
        ],
    )(adj_t, onehot)

    foT = fo.reshape(1, N)
    isoT = iso.reshape(1, N)
    colors = pl.pallas_call(
        _colors_body,
        grid=(G,),
        in_specs=[
            pl.BlockSpec((BI, 1), lambda i: (i, 0)),
            pl.BlockSpec((BI, 1), lambda i: (i, 0)),
            pl.BlockSpec((1, N), lambda i: (0, 0)),
            pl.BlockSpec((1, N), lambda i: (0, 0)),
        ],
        out_specs=pl.BlockSpec((BI, 1), lambda i: (i, 0)),
        out_shape=jax.ShapeDtypeStruct((N, 1), jnp.int32),
    )(fo, iso, foT, isoT)

    return colors.reshape(N).astype(jnp.int64)


# speedup vs baseline: 4.1917x; 1.4145x over previous
import jax
import jax.numpy as jnp
from jax.experimental import pallas as pl
from jax.experimental.pallas import tpu as pltpu

N = 4096
NV = 16
BI = 512
G = N // BI
BJ = 1024
GJ = N // BJ
SD = 40
TSTEPS = G + 1 + G + G


def _rt_dot(a, b):
    return jax.lax.dot_general(
        a, b, (((1,), (1,)), ((), ())), preferred_element_type=jnp.float32
    )


def _fused_body(adj_ref, oh_ref, colors_ref, counts_ref, colsum_ref,
                t_sig_ref, u_sig_ref, fo_ref, iso_ref, foT_ref, isoT_ref):
    t = pl.program_id(0)

    @pl.when(t < G)
    def _():
        adj = adj_ref[...]
        counts_ref[pl.ds(t * BI, BI), :] = jnp.dot(
            adj, oh_ref[...], preferred_element_type=jnp.float32
        )
        part = jnp.sum(adj, axis=0, keepdims=True)

        @pl.when(t == 0)
        def _():
            colsum_ref[...] = part

        @pl.when(t != 0)
        def _():
            colsum_ref[...] = colsum_ref[...] + part

    @pl.when(t == G)
    def _():
        eye = (
            jax.lax.broadcasted_iota(jnp.int32, (BI, BI), 0)
            == jax.lax.broadcasted_iota(jnp.int32, (BI, BI), 1)
        ).astype(jnp.float32)
        cs_rows = [
            _rt_dot(eye, colsum_ref[:, pl.ds(b * BI, BI)]) for b in range(G)
        ]
        colsum_row = jnp.concatenate(cs_rows, axis=0)

        c = counts_ref[...]
        rowsum = jnp.sum(c, axis=1, keepdims=True)
        iso = (rowsum + colsum_row) == 0
        isof = iso.astype(jnp.float32)
        lane = jax.lax.broadcasted_iota(jnp.int32, (N, NV), 1).astype(
            jnp.float32
        )
        xval = jnp.sum(oh_ref[...] * lane, axis=1, keepdims=True)
        ci = c.astype(jnp.int32)
        hi = (ci >> 6).astype(jnp.float32)
        lo = (ci & 63).astype(jnp.float32)
        nrm = isof + xval * xval + jnp.sum(hi * hi + lo * lo, axis=1,
                                           keepdims=True)
        n2 = jnp.floor(nrm * (1.0 / 4096.0)) * 4096.0
        rem = nrm - n2
        n1 = jnp.floor(rem * (1.0 / 64.0)) * 64.0
        n0 = rem - n1
        ones3 = jnp.ones((N, 3), jnp.float32)
        tmat = jnp.concatenate(
            [2 * isof, 2 * xval, 2 * hi, 2 * lo, -n0, -n1, -n2, ones3], axis=1
        )
        umat = jnp.concatenate(
            [isof, xval, hi, lo, ones3, -n0, -n1, -n2], axis=1
        )
        t_sig_ref[...] = tmat.astype(jnp.bfloat16)
        u_sig_ref[...] = umat.astype(jnp.bfloat16)
        iso_ref[...] = iso.astype(jnp.int32)

    @pl.when((t > G) & (t <= 2 * G))
    def _():
        i = t - G - 1
        tb = t_sig_ref[pl.ds(i * BI, BI), :]

        def blockmin(jj):
            g2 = _rt_dot(tb, u_sig_ref[pl.ds(jj * BJ, BJ), :])
            idxj = jax.lax.broadcasted_iota(jnp.int32, (BI, BJ), 1) + jj * BJ
            cand = jnp.where(g2 == 0.0, idxj, jnp.int32(N))
            return jnp.min(cand, axis=1, keepdims=True)

        fo_ref[pl.ds(i * BI, BI), :] = blockmin(0)
        for jj in range(1, GJ):

            @pl.when(jj * BJ <= i * BI)
            def _():
                fo_ref[pl.ds(i * BI, BI), :] = jnp.minimum(
                    fo_ref[pl.ds(i * BI, BI), :], blockmin(jj)
                )

    @pl.when(t > 2 * G)
    def _():
        b = t - 2 * G - 1

        @pl.when(t == 2 * G + 1)
        def _():
            pad8 = jnp.concatenate(
                [
                    fo_ref[...].astype(jnp.float32),
                    iso_ref[...].astype(jnp.float32),
                    jnp.zeros((N, 6), jnp.float32),
                ],
                axis=1,
            )
            sel = (
                jax.lax.broadcasted_iota(jnp.int32, (2, 8), 0)
                == jax.lax.broadcasted_iota(jnp.int32, (2, 8), 1)
            ).astype(jnp.float32)
            both = _rt_dot(sel, pad8)
            foT_ref[...] = both[0:1, :]
            isoT_ref[...] = both[1:2, :]

        fo_b = fo_ref[pl.ds(b * BI, BI), :]
        iso_b = iso_ref[pl.ds(b * BI, BI), :]
        lane = jax.lax.broadcasted_iota(jnp.int32, (1, N), 1).astype(
            jnp.float32
        )
        is_first = (foT_ref[...] == lane) & (isoT_ref[...] == 0.0)
        fo_f = fo_b.astype(jnp.float32)
        le = (
            jax.lax.broadcasted_iota(jnp.int32, (BI, N), 1).astype(jnp.float32)
            <= fo_f
        )
        ranks = jnp.sum((le & is_first).astype(jnp.int32), axis=1,
                        keepdims=True)
        colors_ref[...] = jnp.where(iso_b == 1, 0, ranks)


def kernel(x, adj_t):
    x32 = x.astype(jnp.int32).reshape(N, 1)
    onehot = (x32 == jnp.arange(NV, dtype=jnp.int32)[None, :]).astype(jnp.float32)

    colors = pl.pallas_call(
        _fused_body,
        grid=(TSTEPS,),
        in_specs=[
            pl.BlockSpec((BI, N), lambda t: (jnp.minimum(t, G - 1), 0)),
            pl.BlockSpec((N, NV), lambda t: (0, 0)),
        ],
        out_specs=pl.BlockSpec(
            (BI, 1), lambda t: (jnp.clip(t - 2 * G - 1, 0, G - 1), 0)
        ),
        out_shape=jax.ShapeDtypeStruct((N, 1), jnp.int32),
        scratch_shapes=[
            pltpu.VMEM((N, NV), jnp.float32),
            pltpu.VMEM((1, N), jnp.float32),
            pltpu.VMEM((N, SD), jnp.bfloat16),
            pltpu.VMEM((N, SD), jnp.bfloat16),
            pltpu.VMEM((N, 1), jnp.int32),
            pltpu.VMEM((N, 1), jnp.int32),
            pltpu.VMEM((1, N), jnp.float32),
            pltpu.VMEM((1, N), jnp.float32),
        ],
    )(adj_t, onehot)

    return colors.reshape(N).astype(jnp.int64)
